# baseline (speedup 1.0000x reference)
import functools

import jax
import jax.numpy as jnp
from jax import lax
from jax.experimental import pallas as pl
from jax.experimental.pallas import tpu as pltpu
from jax.experimental.pallas import tpu_sc as plsc

_N = 4096
_L = 16
_NW = 32
_NT = 2048
_NS = _N - _NT
_RPW = _NS // _NW
_BR = 256
_W = 0.1
_BIG = 3.0e38


def _sc_body(xs_h, ys_h, zs_h, cr_h, cg_h, cb_h, bestd_h, smooth_h,
             xs, ys, zs, cr, cg, cb, d2buf, candi, vec16, bestbuf, smbuf):
    cax = lax.axis_index("c")
    sax = lax.axis_index("s")
    wid = sax * 2 + cax
    row0 = _NT + wid * _RPW
    pltpu.sync_copy(xs_h, xs)
    pltpu.sync_copy(ys_h, ys)
    pltpu.sync_copy(zs_h, zs)
    pltpu.sync_copy(cr_h, cr)
    pltpu.sync_copy(cg_h, cg)
    pltpu.sync_copy(cb_h, cb)
    lanes = lax.broadcasted_iota(jnp.int32, (_L,), 0)
    nbmask = (lanes >= 1) & (lanes <= 5)
    bigv = jnp.full((_L,), _BIG, jnp.float32)
    d2buf[pl.ds(_N, _L)] = bigv

    def row_body(r, carry):
        gi = row0 + r
        gsplat = jnp.full((_L,), gi, jnp.int32)
        xi = plsc.load_gather(xs, [gsplat])
        yi = plsc.load_gather(ys, [gsplat])
        zi = plsc.load_gather(zs, [gsplat])

        @plsc.parallel_loop(0, _N, _L, unroll=8, carry=bigv)
        def lane_min(off, m):
            dx = xs[pl.ds(off, _L)] - xi
            dy = ys[pl.ds(off, _L)] - yi
            dz = zs[pl.ds(off, _L)] - zi
            d2 = dx * dx + dy * dy + dz * dz
            d2buf[pl.ds(off, _L)] = d2
            return jnp.minimum(m, d2)
        vec16[...] = lax.sort(lane_min)
        tvec = plsc.load_gather(vec16, [jnp.full((_L,), 5, jnp.int32)])

        @plsc.parallel_loop(0, _N, _L, unroll=8,
                            carry=(jnp.zeros((_L,), jnp.int32), lanes))
        def p2res(off, carry2):
            basev, idxv = carry2
            v = d2buf[pl.ds(off, _L)]
            m = v <= tvec
            mi = jnp.where(m, 1, 0)
            p = jnp.cumsum(mi) - mi
            plsc.store_scatter(candi, [basev + p], idxv, mask=m)
            cnt = plsc.all_reduce_population_count(m)
            return (basev + cnt, idxv + _L)

        basev, _ = p2res
        plsc.store_scatter(candi, [basev + lanes],
                           jnp.full((_L,), _N, jnp.int32) + lanes)
        cc = jnp.max(basev)
        nm = (cc + _L - 1) >> 4

        def pm(k, bvbi):
            bv, bi = bvbi
            ci_ = candi[pl.ds(k * _L, _L)]
            cv = plsc.load_gather(d2buf, [ci_])
            cvs, cis = plsc.sort_key_val(cv, ci_)
            rv = lax.rev(cvs, (0,))
            ri = lax.rev(cis, (0,))
            sel = bv <= rv
            mv = jnp.where(sel, bv, rv)
            mi = jnp.where(sel, bi, ri)
            res = plsc.sort_key_val(mv, mi)
            return (res[0], res[1])

        bv, bi = lax.fori_loop(0, nm, pm,
                               (bigv, jnp.zeros((_L,), jnp.int32)))
        bic = jnp.minimum(bi, _N - 1)
        cri = plsc.load_gather(cr, [gsplat])
        cgi = plsc.load_gather(cg, [gsplat])
        cbi = plsc.load_gather(cb, [gsplat])
        crj = plsc.load_gather(cr, [bic])
        cgj = plsc.load_gather(cg, [bic])
        cbj = plsc.load_gather(cb, [bic])
        diff = (jnp.abs(crj - cri) + jnp.abs(cgj - cgi) + jnp.abs(cbj - cbi))
        smbuf[pl.ds(r * _L, _L)] = jnp.where(nbmask, diff, 0.0)
        bestbuf[pl.ds(r * _L, _L)] = bv
        return carry

    lax.fori_loop(0, _RPW, row_body, 0)
    pltpu.sync_copy(bestbuf, bestd_h.at[pl.ds(wid * _RPW * _L, _RPW * _L)])
    pltpu.sync_copy(smbuf, smooth_h.at[pl.ds(wid * _RPW * _L, _RPW * _L)])


_sc_knn = functools.partial(
    pl.kernel,
    out_type=[jax.ShapeDtypeStruct((_NS * _L,), jnp.float32),
              jax.ShapeDtypeStruct((_NS * _L,), jnp.float32)],
    mesh=plsc.VectorSubcoreMesh(core_axis_name="c", subcore_axis_name="s"),
    compiler_params=pltpu.CompilerParams(needs_layout_passes=False),
    scratch_types=[
        pltpu.VMEM((_N,), jnp.float32),
        pltpu.VMEM((_N,), jnp.float32),
        pltpu.VMEM((_N,), jnp.float32),
        pltpu.VMEM((_N,), jnp.float32),
        pltpu.VMEM((_N,), jnp.float32),
        pltpu.VMEM((_N,), jnp.float32),
        pltpu.VMEM((_N + _L,), jnp.float32),
        pltpu.VMEM((_N + _L,), jnp.int32),
        pltpu.VMEM((_L,), jnp.float32),
        pltpu.VMEM((_RPW * _L,), jnp.float32),
        pltpu.VMEM((_RPW * _L,), jnp.float32),
    ],
)(_sc_body)


def _tc_knn_body(pos_all, col_all, scl_all, rot_all, out_ref):
    step = pl.program_id(0)
    x = pos_all[pl.ds(step * _BR, _BR), :]
    xa = pos_all[...]
    sqi = jnp.sum(x * x, axis=1, keepdims=True)
    sqj = jnp.sum(xa * xa, axis=1)[None, :]
    xy = jax.lax.dot_general(x, xa, (((1,), (1,)), ((), ())),
                             preferred_element_type=jnp.float32)
    d2 = sqi + sqj - 2.0 * xy
    d = jnp.sqrt(jnp.maximum(d2, 1e-12))
    iota = lax.broadcasted_iota(jnp.int32, (_BR, _N), 1)
    rows = step * _BR + lax.broadcasted_iota(jnp.int32, (_BR, _N), 0)
    big = jnp.float32(jnp.inf)
    d = jnp.where(iota == rows, big, d)
    ci = col_all[pl.ds(step * _BR, _BR), :]
    ca = col_all[...]
    sm = jnp.zeros((_BR,), jnp.float32)
    e2 = jnp.zeros((_BR,), jnp.float32)
    for it in range(5):
        m = jnp.min(d, axis=1)
        am = jnp.min(jnp.where(d == m[:, None], iota, _N), axis=1)
        sel = iota == am[:, None]
        oh = sel.astype(jnp.float32)
        cnb = jax.lax.dot_general(oh, ca, (((1,), (0,)), ((), ())),
                                  preferred_element_type=jnp.float32)
        sm = sm + jnp.sum(jnp.abs(ci - cnb), axis=1)
        if it == 1:
            e2 = m
        d = jnp.where(sel, big, d)
    partial = (_W * jnp.sum(jnp.exp(-e2)) + _W * jnp.sum(sm) / 15.0) / _N

    @pl.when(step == 0)
    def _init():
        s = scl_all[...]
        scale_part = jnp.sum(jnp.abs(s - 1.0)) / 3.0
        mu = jnp.mean(s, axis=1, keepdims=True)
        var_part = jnp.sum((s - mu) ** 2) / 2.0
        q = rot_all[...]
        qn = jnp.sqrt(jnp.sum(q * q, axis=1))
        rot_part = jnp.sum((qn - 1.0) ** 2)
        call = col_all[...]
        col_part = jnp.sum((call - 0.5) ** 2) / 3.0
        dense = (_W * (scale_part + var_part) + _W * rot_part
                 + _W * col_part) / _N
        out_ref[...] = jnp.reshape(dense, (1, 1))

    out_ref[...] += jnp.reshape(partial, (1, 1))


def _tc_fin_body(bd_ref, sm_ref, tcp_ref, out_ref):
    bd = bd_ref[...]
    lane = lax.broadcasted_iota(jnp.int32, bd.shape, 1) % _L
    d = jnp.sqrt(jnp.maximum(bd, 1e-12))
    pos_sum = jnp.sum(jnp.where(lane == 2, jnp.exp(-d), 0.0))
    smooth_sum = jnp.sum(sm_ref[...]) / 15.0
    total = (_W * pos_sum + _W * smooth_sum) / _N
    out_ref[...] = tcp_ref[...] + jnp.reshape(total, (1, 1))


def kernel(positions, scales, rotations, colors):
    xs = positions[:, 0]
    ys = positions[:, 1]
    zs = positions[:, 2]
    cr = colors[:, 0]
    cg = colors[:, 1]
    cb = colors[:, 2]
    bestd, smooth = _sc_knn(xs, ys, zs, cr, cg, cb)
    tcpart = pl.pallas_call(
        _tc_knn_body,
        grid=(_NT // _BR,),
        in_specs=[
            pl.BlockSpec((_N, 3), lambda i: (0, 0)),
            pl.BlockSpec((_N, 3), lambda i: (0, 0)),
            pl.BlockSpec((_N, 3), lambda i: (0, 0)),
            pl.BlockSpec((_N, 4), lambda i: (0, 0)),
        ],
        out_specs=pl.BlockSpec((1, 1), lambda i: (0, 0)),
        out_shape=jax.ShapeDtypeStruct((1, 1), jnp.float32),
    )(positions, colors, scales, rotations)
    out = pl.pallas_call(
        _tc_fin_body,
        out_shape=jax.ShapeDtypeStruct((1, 1), jnp.float32),
    )(bestd.reshape(_NS * _L // 128, 128), smooth.reshape(_NS * _L // 128, 128),
      tcpart)
    return out[0, 0]


# speedup vs baseline: 1.1027x; 1.1027x over previous
import functools

import jax
import jax.numpy as jnp
from jax import lax
from jax.experimental import pallas as pl
from jax.experimental.pallas import tpu as pltpu
from jax.experimental.pallas import tpu_sc as plsc

_N = 4096
_L = 16
_NW = 32
_NT = 1792
_NS = _N - _NT
_RPW = _NS // _NW
_BR = 256
_W = 0.1
_BIG = 3.0e38


def _sc_body(xs_h, ys_h, zs_h, cr_h, cg_h, cb_h, bestd_h, smooth_h,
             xs, ys, zs, cr, cg, cb, d2buf, candi, vec16, bestbuf, smbuf,
             dsem):
    cax = lax.axis_index("c")
    sax = lax.axis_index("s")
    wid = sax * 2 + cax
    row0 = _NT + wid * _RPW
    cps = [pltpu.async_copy(s, d, dsem)
           for s, d in ((xs_h, xs), (ys_h, ys), (zs_h, zs),
                        (cr_h, cr), (cg_h, cg), (cb_h, cb))]
    for cp in cps:
        cp.wait()
    lanes = lax.broadcasted_iota(jnp.int32, (_L,), 0)
    nbmask = (lanes >= 1) & (lanes <= 5)
    bigv = jnp.full((_L,), _BIG, jnp.float32)
    d2buf[pl.ds(_N, _L)] = bigv

    def row_body(r, carry):
        gi = row0 + r
        gsplat = jnp.full((_L,), gi, jnp.int32)
        xi = plsc.load_gather(xs, [gsplat])
        yi = plsc.load_gather(ys, [gsplat])
        zi = plsc.load_gather(zs, [gsplat])

        @plsc.parallel_loop(0, _N, _L, unroll=8, carry=bigv)
        def lane_min(off, m):
            dx = xs[pl.ds(off, _L)] - xi
            dy = ys[pl.ds(off, _L)] - yi
            dz = zs[pl.ds(off, _L)] - zi
            d2 = dx * dx + dy * dy + dz * dz
            d2buf[pl.ds(off, _L)] = d2
            return jnp.minimum(m, d2)
        vec16[...] = lax.sort(lane_min)
        tvec = plsc.load_gather(vec16, [jnp.full((_L,), 5, jnp.int32)])

        @plsc.parallel_loop(0, _N, _L, unroll=8,
                            carry=(jnp.zeros((_L,), jnp.int32), lanes))
        def p2res(off, carry2):
            basev, idxv = carry2
            v = d2buf[pl.ds(off, _L)]
            m = v <= tvec
            mi = jnp.where(m, 1, 0)
            p = jnp.cumsum(mi) - mi
            plsc.store_scatter(candi, [basev + p], idxv, mask=m)
            cnt = plsc.all_reduce_population_count(m)
            return (basev + cnt, idxv + _L)

        basev, _ = p2res
        plsc.store_scatter(candi, [basev + lanes],
                           jnp.full((_L,), _N, jnp.int32) + lanes)
        cc = jnp.max(basev)
        nm = (cc + _L - 1) >> 4

        def pm(k, bvbi):
            bv, bi = bvbi
            ci_ = candi[pl.ds(k * _L, _L)]
            cv = plsc.load_gather(d2buf, [ci_])
            cvs, cis = plsc.sort_key_val(cv, ci_)
            rv = lax.rev(cvs, (0,))
            ri = lax.rev(cis, (0,))
            sel = bv <= rv
            mv = jnp.where(sel, bv, rv)
            mi = jnp.where(sel, bi, ri)
            res = plsc.sort_key_val(mv, mi)
            return (res[0], res[1])

        bv, bi = lax.fori_loop(0, nm, pm,
                               (bigv, jnp.zeros((_L,), jnp.int32)))
        bic = jnp.minimum(bi, _N - 1)
        cri = plsc.load_gather(cr, [gsplat])
        cgi = plsc.load_gather(cg, [gsplat])
        cbi = plsc.load_gather(cb, [gsplat])
        crj = plsc.load_gather(cr, [bic])
        cgj = plsc.load_gather(cg, [bic])
        cbj = plsc.load_gather(cb, [bic])
        diff = (jnp.abs(crj - cri) + jnp.abs(cgj - cgi) + jnp.abs(cbj - cbi))
        smbuf[pl.ds(r * _L, _L)] = jnp.where(nbmask, diff, 0.0)
        bestbuf[pl.ds(r * _L, _L)] = bv
        return carry

    lax.fori_loop(0, _RPW, row_body, 0)
    pltpu.sync_copy(bestbuf, bestd_h.at[pl.ds(wid * _RPW * _L, _RPW * _L)])
    pltpu.sync_copy(smbuf, smooth_h.at[pl.ds(wid * _RPW * _L, _RPW * _L)])


_sc_knn = functools.partial(
    pl.kernel,
    out_type=[jax.ShapeDtypeStruct((_NS * _L,), jnp.float32),
              jax.ShapeDtypeStruct((_NS * _L,), jnp.float32)],
    mesh=plsc.VectorSubcoreMesh(core_axis_name="c", subcore_axis_name="s"),
    compiler_params=pltpu.CompilerParams(needs_layout_passes=False),
    scratch_types=[
        pltpu.VMEM((_N,), jnp.float32),
        pltpu.VMEM((_N,), jnp.float32),
        pltpu.VMEM((_N,), jnp.float32),
        pltpu.VMEM((_N,), jnp.float32),
        pltpu.VMEM((_N,), jnp.float32),
        pltpu.VMEM((_N,), jnp.float32),
        pltpu.VMEM((_N + _L,), jnp.float32),
        pltpu.VMEM((_N + _L,), jnp.int32),
        pltpu.VMEM((_L,), jnp.float32),
        pltpu.VMEM((_RPW * _L,), jnp.float32),
        pltpu.VMEM((_RPW * _L,), jnp.float32),
        pltpu.SemaphoreType.DMA,
    ],
)(_sc_body)


def _tc_knn_body(pos_all, col_all, scl_all, rot_all, out_ref):
    step = pl.program_id(0)
    x = pos_all[pl.ds(step * _BR, _BR), :]
    xa = pos_all[...]
    sqi = jnp.sum(x * x, axis=1, keepdims=True)
    sqj = jnp.sum(xa * xa, axis=1)[None, :]
    xy = jax.lax.dot_general(x, xa, (((1,), (1,)), ((), ())),
                             preferred_element_type=jnp.float32)
    d2 = sqi + sqj - 2.0 * xy
    d = jnp.sqrt(jnp.maximum(d2, 1e-12))
    iota = lax.broadcasted_iota(jnp.int32, (_BR, _N), 1)
    rows = step * _BR + lax.broadcasted_iota(jnp.int32, (_BR, _N), 0)
    big = jnp.float32(jnp.inf)
    d = jnp.where(iota == rows, big, d)
    ci = col_all[pl.ds(step * _BR, _BR), :]
    ca = col_all[...]
    sm = jnp.zeros((_BR,), jnp.float32)
    e2 = jnp.zeros((_BR,), jnp.float32)
    for it in range(5):
        m = jnp.min(d, axis=1)
        am = jnp.min(jnp.where(d == m[:, None], iota, _N), axis=1)
        sel = iota == am[:, None]
        oh = sel.astype(jnp.float32)
        cnb = jax.lax.dot_general(oh, ca, (((1,), (0,)), ((), ())),
                                  preferred_element_type=jnp.float32)
        sm = sm + jnp.sum(jnp.abs(ci - cnb), axis=1)
        if it == 1:
            e2 = m
        d = jnp.where(sel, big, d)
    partial = (_W * jnp.sum(jnp.exp(-e2)) + _W * jnp.sum(sm) / 15.0) / _N

    @pl.when(step == 0)
    def _init():
        s = scl_all[...]
        scale_part = jnp.sum(jnp.abs(s - 1.0)) / 3.0
        mu = jnp.mean(s, axis=1, keepdims=True)
        var_part = jnp.sum((s - mu) ** 2) / 2.0
        q = rot_all[...]
        qn = jnp.sqrt(jnp.sum(q * q, axis=1))
        rot_part = jnp.sum((qn - 1.0) ** 2)
        call = col_all[...]
        col_part = jnp.sum((call - 0.5) ** 2) / 3.0
        dense = (_W * (scale_part + var_part) + _W * rot_part
                 + _W * col_part) / _N
        out_ref[...] = jnp.reshape(dense, (1, 1))

    out_ref[...] += jnp.reshape(partial, (1, 1))


def _tc_fin_body(bd_ref, sm_ref, tcp_ref, out_ref):
    bd = bd_ref[...]
    lane = lax.broadcasted_iota(jnp.int32, bd.shape, 1) % _L
    d = jnp.sqrt(jnp.maximum(bd, 1e-12))
    pos_sum = jnp.sum(jnp.where(lane == 2, jnp.exp(-d), 0.0))
    smooth_sum = jnp.sum(sm_ref[...]) / 15.0
    total = (_W * pos_sum + _W * smooth_sum) / _N
    out_ref[...] = tcp_ref[...] + jnp.reshape(total, (1, 1))


def kernel(positions, scales, rotations, colors):
    xs = positions[:, 0]
    ys = positions[:, 1]
    zs = positions[:, 2]
    cr = colors[:, 0]
    cg = colors[:, 1]
    cb = colors[:, 2]
    bestd, smooth = _sc_knn(xs, ys, zs, cr, cg, cb)
    tcpart = pl.pallas_call(
        _tc_knn_body,
        grid=(_NT // _BR,),
        in_specs=[
            pl.BlockSpec((_N, 3), lambda i: (0, 0)),
            pl.BlockSpec((_N, 3), lambda i: (0, 0)),
            pl.BlockSpec((_N, 3), lambda i: (0, 0)),
            pl.BlockSpec((_N, 4), lambda i: (0, 0)),
        ],
        out_specs=pl.BlockSpec((1, 1), lambda i: (0, 0)),
        out_shape=jax.ShapeDtypeStruct((1, 1), jnp.float32),
    )(positions, colors, scales, rotations)
    out = pl.pallas_call(
        _tc_fin_body,
        out_shape=jax.ShapeDtypeStruct((1, 1), jnp.float32),
    )(bestd.reshape(_NS * _L // 128, 128), smooth.reshape(_NS * _L // 128, 128),
      tcpart)
    return out[0, 0]


# speedup vs baseline: 1.1048x; 1.0019x over previous
import functools

import jax
import jax.numpy as jnp
from jax import lax
from jax.experimental import pallas as pl
from jax.experimental.pallas import tpu as pltpu
from jax.experimental.pallas import tpu_sc as plsc

_N = 4096
_L = 16
_NW = 32
_NT = 1792
_NS = _N - _NT
_RPW = _NS // _NW
_BR = 256
_W = 0.1
_BIG = 3.0e38
_CAP = 256


def _sc_body(xs_h, ys_h, zs_h, cr_h, cg_h, cb_h, bestd_h, smooth_h,
             xs, ys, zs, cr, cg, cb, d2buf, candi, vec16, bestbuf, smbuf,
             dsem):
    cax = lax.axis_index("c")
    sax = lax.axis_index("s")
    wid = sax * 2 + cax
    row0 = _NT + wid * _RPW
    cps = [pltpu.async_copy(s, d, dsem)
           for s, d in ((xs_h, xs), (ys_h, ys), (zs_h, zs),
                        (cr_h, cr), (cg_h, cg), (cb_h, cb))]
    for cp in cps:
        cp.wait()
    lanes = lax.broadcasted_iota(jnp.int32, (_L,), 0)
    nbmask = (lanes >= 1) & (lanes <= 5)
    bigv = jnp.full((_L,), _BIG, jnp.float32)
    d2buf[pl.ds(_N, _L)] = bigv

    def row_body(r, carry):
        gi = row0 + r
        gsplat = jnp.full((_L,), gi, jnp.int32)
        xi = plsc.load_gather(xs, [gsplat])
        yi = plsc.load_gather(ys, [gsplat])
        zi = plsc.load_gather(zs, [gsplat])

        @plsc.parallel_loop(0, _N, _L, unroll=8, carry=bigv)
        def lane_min(off, m):
            dx = xs[pl.ds(off, _L)] - xi
            dy = ys[pl.ds(off, _L)] - yi
            dz = zs[pl.ds(off, _L)] - zi
            d2 = dx * dx + dy * dy + dz * dz
            d2buf[pl.ds(off, _L)] = d2
            return jnp.minimum(m, d2)
        vec16[...] = lax.sort(lane_min)
        tvec = plsc.load_gather(vec16, [jnp.full((_L,), 5, jnp.int32)])

        laneoff = lanes * (_CAP + 1)
        ones16 = jnp.full((_L,), 1, jnp.int32)

        @plsc.parallel_loop(0, _N, _L, unroll=8,
                            carry=(jnp.zeros((_L,), jnp.int32), lanes))
        def p2res(off, carry2):
            cnts, idxv = carry2
            v = d2buf[pl.ds(off, _L)]
            m = v <= tvec
            plsc.store_scatter(candi, [laneoff + cnts], idxv, mask=m)
            return (cnts + jnp.where(m, 1, 0), idxv + _L)

        cnts, _ = p2res
        plsc.store_scatter(candi, [laneoff + cnts],
                           jnp.full((_L,), _N, jnp.int32))
        nmax = jnp.max(cnts) + 1

        def pm(k, bvbi):
            bv, bi = bvbi
            kv = jnp.full((_L,), k, jnp.int32)
            ci_ = plsc.load_gather(candi, [laneoff + kv])
            ci_ = jnp.where(kv <= cnts, ci_, _N)
            cv = plsc.load_gather(d2buf, [ci_])
            cvs, cis = plsc.sort_key_val(cv, ci_)
            rv = lax.rev(cvs, (0,))
            ri = lax.rev(cis, (0,))
            sel = bv <= rv
            mv = jnp.where(sel, bv, rv)
            mi = jnp.where(sel, bi, ri)
            res = plsc.sort_key_val(mv, mi)
            return (res[0], res[1])

        bv, bi = lax.fori_loop(0, nmax, pm,
                               (bigv, jnp.zeros((_L,), jnp.int32)))
        bic = jnp.minimum(bi, _N - 1)
        cri = plsc.load_gather(cr, [gsplat])
        cgi = plsc.load_gather(cg, [gsplat])
        cbi = plsc.load_gather(cb, [gsplat])
        crj = plsc.load_gather(cr, [bic])
        cgj = plsc.load_gather(cg, [bic])
        cbj = plsc.load_gather(cb, [bic])
        diff = (jnp.abs(crj - cri) + jnp.abs(cgj - cgi) + jnp.abs(cbj - cbi))
        smbuf[pl.ds(r * _L, _L)] = jnp.where(nbmask, diff, 0.0)
        bestbuf[pl.ds(r * _L, _L)] = bv
        return carry

    lax.fori_loop(0, _RPW, row_body, 0)
    pltpu.sync_copy(bestbuf, bestd_h.at[pl.ds(wid * _RPW * _L, _RPW * _L)])
    pltpu.sync_copy(smbuf, smooth_h.at[pl.ds(wid * _RPW * _L, _RPW * _L)])


_sc_knn = functools.partial(
    pl.kernel,
    out_type=[jax.ShapeDtypeStruct((_NS * _L,), jnp.float32),
              jax.ShapeDtypeStruct((_NS * _L,), jnp.float32)],
    mesh=plsc.VectorSubcoreMesh(core_axis_name="c", subcore_axis_name="s"),
    compiler_params=pltpu.CompilerParams(needs_layout_passes=False),
    scratch_types=[
        pltpu.VMEM((_N,), jnp.float32),
        pltpu.VMEM((_N,), jnp.float32),
        pltpu.VMEM((_N,), jnp.float32),
        pltpu.VMEM((_N,), jnp.float32),
        pltpu.VMEM((_N,), jnp.float32),
        pltpu.VMEM((_N,), jnp.float32),
        pltpu.VMEM((_N + _L,), jnp.float32),
        pltpu.VMEM((_L * (_CAP + 1),), jnp.int32),
        pltpu.VMEM((_L,), jnp.float32),
        pltpu.VMEM((_RPW * _L,), jnp.float32),
        pltpu.VMEM((_RPW * _L,), jnp.float32),
        pltpu.SemaphoreType.DMA,
    ],
)(_sc_body)


def _tc_knn_body(pos_all, col_all, scl_all, rot_all, out_ref):
    step = pl.program_id(0)
    x = pos_all[pl.ds(step * _BR, _BR), :]
    xa = pos_all[...]
    sqi = jnp.sum(x * x, axis=1, keepdims=True)
    sqj = jnp.sum(xa * xa, axis=1)[None, :]
    xy = jax.lax.dot_general(x, xa, (((1,), (1,)), ((), ())),
                             preferred_element_type=jnp.float32)
    d2 = sqi + sqj - 2.0 * xy
    d = jnp.sqrt(jnp.maximum(d2, 1e-12))
    iota = lax.broadcasted_iota(jnp.int32, (_BR, _N), 1)
    rows = step * _BR + lax.broadcasted_iota(jnp.int32, (_BR, _N), 0)
    big = jnp.float32(jnp.inf)
    d = jnp.where(iota == rows, big, d)
    ci = col_all[pl.ds(step * _BR, _BR), :]
    ca = col_all[...]
    sm = jnp.zeros((_BR,), jnp.float32)
    e2 = jnp.zeros((_BR,), jnp.float32)
    for it in range(5):
        m = jnp.min(d, axis=1)
        am = jnp.min(jnp.where(d == m[:, None], iota, _N), axis=1)
        sel = iota == am[:, None]
        oh = sel.astype(jnp.float32)
        cnb = jax.lax.dot_general(oh, ca, (((1,), (0,)), ((), ())),
                                  preferred_element_type=jnp.float32)
        sm = sm + jnp.sum(jnp.abs(ci - cnb), axis=1)
        if it == 1:
            e2 = m
        d = jnp.where(sel, big, d)
    partial = (_W * jnp.sum(jnp.exp(-e2)) + _W * jnp.sum(sm) / 15.0) / _N

    @pl.when(step == 0)
    def _init():
        s = scl_all[...]
        scale_part = jnp.sum(jnp.abs(s - 1.0)) / 3.0
        mu = jnp.mean(s, axis=1, keepdims=True)
        var_part = jnp.sum((s - mu) ** 2) / 2.0
        q = rot_all[...]
        qn = jnp.sqrt(jnp.sum(q * q, axis=1))
        rot_part = jnp.sum((qn - 1.0) ** 2)
        call = col_all[...]
        col_part = jnp.sum((call - 0.5) ** 2) / 3.0
        dense = (_W * (scale_part + var_part) + _W * rot_part
                 + _W * col_part) / _N
        out_ref[...] = jnp.reshape(dense, (1, 1))

    out_ref[...] += jnp.reshape(partial, (1, 1))


def _tc_fin_body(bd_ref, sm_ref, tcp_ref, out_ref):
    bd = bd_ref[...]
    lane = lax.broadcasted_iota(jnp.int32, bd.shape, 1) % _L
    d = jnp.sqrt(jnp.maximum(bd, 1e-12))
    pos_sum = jnp.sum(jnp.where(lane == 2, jnp.exp(-d), 0.0))
    smooth_sum = jnp.sum(sm_ref[...]) / 15.0
    total = (_W * pos_sum + _W * smooth_sum) / _N
    out_ref[...] = tcp_ref[...] + jnp.reshape(total, (1, 1))


def kernel(positions, scales, rotations, colors):
    xs = positions[:, 0]
    ys = positions[:, 1]
    zs = positions[:, 2]
    cr = colors[:, 0]
    cg = colors[:, 1]
    cb = colors[:, 2]
    bestd, smooth = _sc_knn(xs, ys, zs, cr, cg, cb)
    tcpart = pl.pallas_call(
        _tc_knn_body,
        grid=(_NT // _BR,),
        in_specs=[
            pl.BlockSpec((_N, 3), lambda i: (0, 0)),
            pl.BlockSpec((_N, 3), lambda i: (0, 0)),
            pl.BlockSpec((_N, 3), lambda i: (0, 0)),
            pl.BlockSpec((_N, 4), lambda i: (0, 0)),
        ],
        out_specs=pl.BlockSpec((1, 1), lambda i: (0, 0)),
        out_shape=jax.ShapeDtypeStruct((1, 1), jnp.float32),
    )(positions, colors, scales, rotations)
    out = pl.pallas_call(
        _tc_fin_body,
        out_shape=jax.ShapeDtypeStruct((1, 1), jnp.float32),
    )(bestd.reshape(_NS * _L // 128, 128), smooth.reshape(_NS * _L // 128, 128),
      tcpart)
    return out[0, 0]
